# baseline (speedup 1.0000x reference)
import functools

import jax
import jax.numpy as jnp
from jax import lax
from jax.experimental import pallas as pl
from jax.experimental.pallas import tpu as pltpu
from jax.experimental.pallas import tpu_sc as plsc

B, T, D = 1024, 50, 64
DP = 2 * D
V_ANS = 1000
NC, NS, L = 2, 16, 16
NW = NC * NS
BPW = B // NW
NRING = 4
NJ = BPW // NRING
NV = T // L + 1


@functools.partial(
    pl.kernel,
    mesh=plsc.VectorSubcoreMesh(core_axis_name="c", subcore_axis_name="s"),
    out_type=jax.ShapeDtypeStruct((B, T, D), jnp.float32),
    scratch_types=(
        [pltpu.VMEM((BPW * T,), jnp.int32)]
        + [pltpu.VMEM((NV * L, DP), jnp.float32)] * NRING
        + [pltpu.VMEM((T, D), jnp.float32)] * NRING
        + [pltpu.VMEM((T, D), jnp.float32)] * NRING
        + [pltpu.VMEM((NV * L,), jnp.int32)] * NRING
        + [pltpu.SemaphoreType.DMA]
        + [pltpu.SemaphoreType.DMA] * NRING
        + [pltpu.SemaphoreType.DMA] * NRING
        + [pltpu.SemaphoreType.DMA] * NRING
    ),
    compiler_params=pltpu.CompilerParams(needs_layout_passes=False),
)
def _gather_kernel(
    ans_hbm, ocr_hbm, inds_hbm, out_hbm,
    idx_v,
    st0, st1, st2, st3, sl0, sl1, sl2, sl3,
    ob0, ob1, ob2, ob3, li0, li1, li2, li3,
    sem_iv, sg0, sg1, sg2, sg3, so0, so1, so2, so3,
    su0, su1, su2, su3,
):
    stage = (st0, st1, st2, st3)
    slot = (sl0, sl1, sl2, sl3)
    obuf = (ob0, ob1, ob2, ob3)
    lst = (li0, li1, li2, li3)
    sg = (sg0, sg1, sg2, sg3)
    so = (so0, so1, so2, so3)
    su = (su0, su1, su2, su3)

    wid = lax.axis_index("s") * NC + lax.axis_index("c")
    b0 = wid * BPW

    cp_iv = pltpu.async_copy(
        inds_hbm.at[pl.ds(b0 * T, BPW * T)], idx_v, sem_iv
    )
    for s in range(NRING):
        pltpu.async_copy(ocr_hbm.at[b0 + s], slot[s], so[s])
    cp_iv.wait()

    def build_list_and_gather(i, s):
        for q in range(NV):
            ids = jnp.minimum(
                lax.iota(jnp.int32, L) + (i * T + L * q), BPW * T - 1
            )
            v = plsc.load_gather(idx_v, [ids])
            lst[s][pl.ds(L * q, L)] = jnp.minimum(v, V_ANS - 1)
        pltpu.async_copy(
            ans_hbm.at[lst[s].at[pl.ds(0, T)]], stage[s].at[pl.ds(0, T)], sg[s]
        )

    build_list_and_gather(0, 0)
    build_list_and_gather(1, 1)

    def ring_step(jj, carry):
        for s in range(NRING):
            i = jj * NRING + s
            b = b0 + i
            pltpu.make_async_copy(
                ans_hbm.at[lst[s].at[pl.ds(0, T)]],
                stage[s].at[pl.ds(0, T)], sg[s],
            ).wait()
            pltpu.make_async_copy(ocr_hbm.at[0], slot[s], so[s]).wait()

            @pl.when(jj > 0)
            def _():
                pltpu.make_async_copy(obuf[s], out_hbm.at[0], su[s]).wait()

            for g in range(NV):
                ids = jnp.minimum(
                    lax.iota(jnp.int32, L) + (i * T + L * g), BPW * T - 1
                )
                vg = plsc.load_gather(idx_v, [ids])
                for rl in range(min(L, T - g * L)):
                    r = g * L + rl
                    for q in range(D // L):
                        obuf[s][r, pl.ds(L * q, L)] = stage[s][r, pl.ds(L * q, L)]
                    val = vg[rl]

                    @pl.when(val >= V_ANS)
                    def _():
                        k = val - V_ANS
                        for q in range(D // L):
                            obuf[s][r, pl.ds(L * q, L)] = slot[s][k, pl.ds(L * q, L)]

            pltpu.async_copy(obuf[s], out_hbm.at[b], su[s])

            @pl.when(jj < NJ - 1)
            def _():
                pltpu.async_copy(ocr_hbm.at[b + NRING], slot[s], so[s])

            s2 = (s + 2) % NRING
            if s < 2:
                build_list_and_gather(i + 2, s2)
            else:
                @pl.when(jj < NJ - 1)
                def _():
                    build_list_and_gather(i + 2, s2)
        return carry

    lax.fori_loop(0, NJ, ring_step, 0)
    for s in range(NRING):
        pltpu.make_async_copy(obuf[s], out_hbm.at[0], su[s]).wait()


def kernel(ans_emb, ocr_emb, prev_inds):
    ans_p = jnp.pad(ans_emb, ((0, 0), (0, DP - D)))
    inds1 = prev_inds.astype(jnp.int32).reshape(-1)
    return _gather_kernel(ans_p, ocr_emb, inds1)


# speedup vs baseline: 1.9382x; 1.9382x over previous
import functools

import jax
import jax.numpy as jnp
from jax import lax
from jax.experimental import pallas as pl
from jax.experimental.pallas import tpu as pltpu
from jax.experimental.pallas import tpu_sc as plsc

B, T, D = 1024, 50, 64
V_ANS = 1000
V_TAB = V_ANS + 2 * T
NC, NS, L = 2, 16, 16
NW = NC * NS
BPW = B // NW
NPAIR = BPW // 2
ROW_W = T * D


@functools.partial(
    pl.kernel,
    mesh=plsc.VectorSubcoreMesh(core_axis_name="c", subcore_axis_name="s"),
    out_type=jax.ShapeDtypeStruct((B * T * D,), jnp.float32),
    scratch_types=[
        pltpu.VMEM((V_TAB * D,), jnp.float32),
        pltpu.VMEM((BPW * T,), jnp.int32),
        pltpu.VMEM((ROW_W,), jnp.float32),
        pltpu.VMEM((ROW_W,), jnp.float32),
        pltpu.SemaphoreType.DMA,
        pltpu.SemaphoreType.DMA,
        pltpu.SemaphoreType.DMA,
        pltpu.SemaphoreType.DMA,
        pltpu.SemaphoreType.DMA,
        pltpu.SemaphoreType.DMA,
    ],
    compiler_params=pltpu.CompilerParams(
        needs_layout_passes=False, use_tc_tiling_on_sc=False
    ),
)
def _gather_kernel(
    ans_hbm, ocr_hbm, inds_hbm, out_hbm,
    table, idx_all, out0, out1,
    sem_ans, sem_idx, so0, so1, su0, su1,
):
    wid = lax.axis_index("s") * NC + lax.axis_index("c")
    b0 = wid * BPW

    cp_ans = pltpu.async_copy(
        ans_hbm, table.at[pl.ds(0, V_ANS * D)], sem_ans
    )
    cp_idx = pltpu.async_copy(
        inds_hbm.at[pl.ds(b0 * T, BPW * T)], idx_all, sem_idx
    )
    pltpu.async_copy(
        ocr_hbm.at[pl.ds(b0 * ROW_W, ROW_W)],
        table.at[pl.ds(V_ANS * D, ROW_W)], so0,
    )
    pltpu.async_copy(
        ocr_hbm.at[pl.ds((b0 + 1) * ROW_W, ROW_W)],
        table.at[pl.ds((V_ANS + T) * D, ROW_W)], so1,
    )
    cp_idx.wait()
    cp_ans.wait()

    def do_batch(j, i, slot, out_buf, sem_o, sem_u):
        slot_ds = pl.ds((V_ANS + T * slot) * D, ROW_W)
        pltpu.make_async_copy(
            ocr_hbm.at[pl.ds(0, ROW_W)], table.at[slot_ds], sem_o
        ).wait()

        @pl.when(j > 0)
        def _():
            pltpu.make_async_copy(
                out_buf, out_hbm.at[pl.ds(0, ROW_W)], sem_u
            ).wait()

        @functools.partial(plsc.parallel_loop, 0, T, unroll=8)
        def _(r):
            row = plsc.load_gather(
                idx_all, [jnp.full((L,), i * T + r, jnp.int32)]
            )
            if slot == 1:
                row = jnp.where(row >= V_ANS, row + T, row)
            base = row * D
            for q in range(D // L):
                col = lax.iota(jnp.int32, L) + (L * q)
                out_buf[pl.ds(r * D + L * q, L)] = plsc.load_gather(
                    table, [base + col]
                )

        pltpu.async_copy(out_buf, out_hbm.at[pl.ds((b0 + i) * ROW_W, ROW_W)], sem_u)

        @pl.when(j < NPAIR - 1)
        def _():
            pltpu.async_copy(
                ocr_hbm.at[pl.ds((b0 + i + 2) * ROW_W, ROW_W)],
                table.at[slot_ds], sem_o,
            )

    def pair_step(j, carry):
        do_batch(j, 2 * j, 0, out0, so0, su0)
        do_batch(j, 2 * j + 1, 1, out1, so1, su1)
        return carry

    lax.fori_loop(0, NPAIR, pair_step, 0)
    pltpu.make_async_copy(out0, out_hbm.at[pl.ds(0, ROW_W)], su0).wait()
    pltpu.make_async_copy(out1, out_hbm.at[pl.ds(0, ROW_W)], su1).wait()


def kernel(ans_emb, ocr_emb, prev_inds):
    ans1 = ans_emb.reshape(-1)
    ocr1 = ocr_emb.reshape(-1)
    inds1 = prev_inds.astype(jnp.int32).reshape(-1)
    out1 = _gather_kernel(ans1, ocr1, inds1)
    return out1.reshape(B, T, D)
